# baseline (speedup 1.0000x reference)
import jax
import jax.numpy as jnp
from jax import lax
from jax.experimental import pallas as pl
from jax.experimental.pallas import tpu as pltpu
from jax.experimental.pallas import tpu_sc as plsc

_B = 16384
_D = 128
_NE = 1000
_PARP = 3072

_NC = 2
_NS = 16
_NW = _NC * _NS
_BPW = _B // _NW
_CHUNKS = _BPW // 16


def _body(z_hbm, t_hbm, e_hbm, par_hbm, out_hbm,
          zsh, parsh, t_v, e_v, zidx_v, pidx_v, zsel_v, psel_v,
          logit_v, sem, semg, semz0, semz1, semz2, semz3):
    s = lax.axis_index("s")
    wid = lax.axis_index("c") * _NS + s
    base = wid * _BPW
    sbase = s * _BPW * _D
    gt = pltpu.async_copy(t_hbm.at[pl.ds(base, _BPW)], t_v, sem)
    ge = pltpu.async_copy(e_hbm.at[pl.ds(base, _BPW)], e_v, sem)
    _ZC = _BPW // 4 * _D
    zsems = (semz0, semz1, semz2, semz3)
    zst = []
    for k in range(4):
        zst.append(pltpu.async_copy(
            z_hbm.at[pl.ds(base * _D + k * _ZC, _ZC)],
            zsh.at[pl.ds(sbase + k * _ZC, _ZC)], zsems[k]))
    gpar = pltpu.async_copy(par_hbm, parsh.at[pl.ds(s * _PARP, _PARP)], sem)
    gt.wait()
    ge.wait()
    iota = lax.iota(jnp.int32, 16)
    pb = s * _PARP
    for j in range(_CHUNKS):
        t16 = t_v[pl.ds(j * 16, 16)]
        loc16 = iota + (j * 16)
        zidx_v[pl.ds(j * 16, 16)] = sbase + loc16 * _D + t16
        e16 = e_v[pl.ds(j * 16, 16)] + pb
        pidx_v[pl.ds(j * 16, 16)] = e16
        pidx_v[pl.ds(_BPW + j * 16, 16)] = e16 + _NE
        pidx_v[pl.ds(2 * _BPW + j * 16, 16)] = e16 + 2 * _NE
    gpar.wait()
    gp = pltpu.async_copy(parsh.at[pidx_v], psel_v, semg)
    gzs = []
    for k in range(4):
        zst[k].wait()
        gzs.append(pltpu.async_copy(
            zsh.at[zidx_v.at[pl.ds(k * (_BPW // 4), _BPW // 4)]],
            zsel_v.at[pl.ds(k * (_BPW // 4), _BPW // 4)], semg))
    gp.wait()
    for g in gzs:
        g.wait()
    for j in range(_CHUNKS):
        zs = zsel_v[pl.ds(j * 16, 16)]
        ic = psel_v[pl.ds(j * 16, 16)]
        sh = psel_v[pl.ds(_BPW + j * 16, 16)]
        la = psel_v[pl.ds(2 * _BPW + j * 16, 16)]
        zl = zs * la
        logit_v[pl.ds(j * 16, 16)] = sh + zs * ic - zl * zl
    pltpu.sync_copy(logit_v, out_hbm.at[pl.ds(base, _BPW)])


def kernel(z, t, env_ids, intercepts, shifts, lambdas):
    t32 = t.astype(jnp.int32)
    e32 = env_ids.astype(jnp.int32)
    par = jnp.concatenate([intercepts, shifts, lambdas,
                           jnp.zeros((_PARP - 3 * _NE,), jnp.float32)])
    mesh = plsc.VectorSubcoreMesh(core_axis_name="c", subcore_axis_name="s")
    f = pl.kernel(
        _body,
        mesh=mesh,
        out_type=jax.ShapeDtypeStruct((_B,), jnp.float32),
        scratch_types=[
            pltpu.VMEM_SHARED((_NS * _BPW * _D,), jnp.float32),
            pltpu.VMEM_SHARED((_NS * _PARP,), jnp.float32),
            pltpu.VMEM((_BPW,), jnp.int32),
            pltpu.VMEM((_BPW,), jnp.int32),
            pltpu.VMEM((_BPW,), jnp.int32),
            pltpu.VMEM((3 * _BPW,), jnp.int32),
            pltpu.VMEM((_BPW,), jnp.float32),
            pltpu.VMEM((3 * _BPW,), jnp.float32),
            pltpu.VMEM((_BPW,), jnp.float32),
            pltpu.SemaphoreType.DMA,
            pltpu.SemaphoreType.DMA,
            pltpu.SemaphoreType.DMA,
            pltpu.SemaphoreType.DMA,
            pltpu.SemaphoreType.DMA,
            pltpu.SemaphoreType.DMA,
        ],
    )
    lg = f(z.reshape(_B * _D), t32, e32, par)
    return jnp.concatenate([jnp.zeros((_B, 1), jnp.float32),
                            lg.reshape(_B, 1)], axis=1)


# speedup vs baseline: 1.0233x; 1.0233x over previous
import jax
import jax.numpy as jnp
from jax import lax
from jax.experimental import pallas as pl
from jax.experimental.pallas import tpu as pltpu
from jax.experimental.pallas import tpu_sc as plsc

_B = 16384
_D = 128
_NE = 1000
_PARP = 3072

_NC = 2
_NS = 16
_NW = _NC * _NS
_BPW = _B // _NW
_CHUNKS = _BPW // 16


def _body(z_hbm, t_hbm, e_hbm, par_hbm, out_hbm,
          zsh, parsh, t_v, e_v, zidx_v, pidx_v, zsel_v, psel_v,
          logit_v, sem):
    s = lax.axis_index("s")
    wid = lax.axis_index("c") * _NS + s
    base = wid * _BPW
    sbase = s * _BPW * _D
    gr = pltpu.async_copy(z_hbm.at[pl.ds(base * _D, _BPW * _D)],
                          zsh.at[pl.ds(sbase, _BPW * _D)], sem)
    gpar = pltpu.async_copy(par_hbm, parsh.at[pl.ds(s * _PARP, _PARP)], sem)
    pltpu.sync_copy(t_hbm.at[pl.ds(base, _BPW)], t_v)
    pltpu.sync_copy(e_hbm.at[pl.ds(base, _BPW)], e_v)
    iota = lax.iota(jnp.int32, 16)
    pb = s * _PARP

    def idx_body(j, _):
        o = pl.multiple_of(j * 16, 16)
        t16 = t_v[pl.ds(o, 16)]
        loc16 = iota + o
        zidx_v[pl.ds(o, 16)] = sbase + loc16 * _D + t16
        e16 = e_v[pl.ds(o, 16)] + pb
        pidx_v[pl.ds(o, 16)] = e16
        pidx_v[pl.ds(_BPW + o, 16)] = e16 + _NE
        pidx_v[pl.ds(2 * _BPW + o, 16)] = e16 + 2 * _NE
        return 0

    lax.fori_loop(0, _CHUNKS, idx_body, 0)
    gr.wait()
    gpar.wait()
    gz = pltpu.async_copy(zsh.at[zidx_v], zsel_v, sem)
    gp = pltpu.async_copy(parsh.at[pidx_v], psel_v, sem)
    gz.wait()
    gp.wait()
    def comp_body(j, _):
        o = pl.multiple_of(j * 16, 16)
        zs = zsel_v[pl.ds(o, 16)]
        ic = psel_v[pl.ds(o, 16)]
        sh = psel_v[pl.ds(_BPW + o, 16)]
        la = psel_v[pl.ds(2 * _BPW + o, 16)]
        zl = zs * la
        logit_v[pl.ds(o, 16)] = sh + zs * ic - zl * zl
        return 0

    lax.fori_loop(0, _CHUNKS, comp_body, 0)
    pltpu.sync_copy(logit_v, out_hbm.at[pl.ds(base, _BPW)])


def kernel(z, t, env_ids, intercepts, shifts, lambdas):
    t32 = t.astype(jnp.int32)
    e32 = env_ids.astype(jnp.int32)
    par = jnp.concatenate([intercepts, shifts, lambdas,
                           jnp.zeros((_PARP - 3 * _NE,), jnp.float32)])
    mesh = plsc.VectorSubcoreMesh(core_axis_name="c", subcore_axis_name="s")
    f = pl.kernel(
        _body,
        mesh=mesh,
        out_type=jax.ShapeDtypeStruct((_B,), jnp.float32),
        scratch_types=[
            pltpu.VMEM_SHARED((_NS * _BPW * _D,), jnp.float32),
            pltpu.VMEM_SHARED((_NS * _PARP,), jnp.float32),
            pltpu.VMEM((_BPW,), jnp.int32),
            pltpu.VMEM((_BPW,), jnp.int32),
            pltpu.VMEM((_BPW,), jnp.int32),
            pltpu.VMEM((3 * _BPW,), jnp.int32),
            pltpu.VMEM((_BPW,), jnp.float32),
            pltpu.VMEM((3 * _BPW,), jnp.float32),
            pltpu.VMEM((_BPW,), jnp.float32),
            pltpu.SemaphoreType.DMA,
        ],
    )
    lg = f(z.reshape(_B * _D), t32, e32, par)
    return jnp.concatenate([jnp.zeros((_B, 1), jnp.float32),
                            lg.reshape(_B, 1)], axis=1)
